# Initial kernel scaffold; baseline (speedup 1.0000x reference)
import jax
import jax.numpy as jnp
from jax.experimental import pallas as pl


def kernel(x, edge_index, W1, b1, W2, b2, W3, b3):
    raise NotImplementedError("write your pallas kernel here")


# speedup vs baseline: 15.9553x; 15.9553x over previous
import functools

import jax
import jax.numpy as jnp
from jax import lax
from jax.experimental import pallas as pl
from jax.experimental.pallas import tpu as pltpu
from jax.experimental.pallas import tpu_sc as plsc

N = 10000
E = 320000
D = 128
NC = 2
NS = 16
NPAD = 10240
TRASH = N
EPT = E // (NC * NS)
K = 80
CHUNKS = EPT // K
RPT = NPAD // NS

_mesh = plsc.VectorSubcoreMesh(
    core_axis_name="c", subcore_axis_name="s", num_cores=NC, num_subcores=NS
)


def _edge_chunk_indices(row_all, col_all, rowbuf, colbuf, off):
    trash = jnp.full((16,), TRASH, jnp.int32)
    for j in range(K // 16):
        r = row_all[pl.ds(off + j * 16, 16)]
        c = col_all[pl.ds(off + j * 16, 16)]
        rowbuf[pl.ds(j * 16, 16)] = r
        colbuf[pl.ds(j * 16, 16)] = jnp.where(r == c, trash, c)


@functools.partial(
    pl.kernel,
    out_type=jax.ShapeDtypeStruct((NC, NPAD, D), jnp.float32),
    mesh=_mesh,
    scratch_types=[
        pltpu.VMEM((K,), jnp.int32),
        pltpu.VMEM((K,), jnp.int32),
        pltpu.VMEM((EPT,), jnp.int32),
        pltpu.VMEM((EPT,), jnp.int32),
        pltpu.VMEM((K, D), jnp.float32),
        pltpu.VMEM_SHARED((NPAD, D), jnp.float32),
        pltpu.SemaphoreType.DMA,
    ],
)
def _sc_agg(zp, row, col, out, rowbuf, colbuf, row_all, col_all, msg, acc, sem):
    c = lax.axis_index("c")
    s = lax.axis_index("s")

    def _zrow(r, carry):
        for j in range(D // 16):
            msg[r, pl.ds(j * 16, 16)] = jnp.zeros((16,), jnp.float32)
        return carry

    lax.fori_loop(0, K, _zrow, 0)
    base = s * RPT
    for k in range(RPT // K):
        pltpu.sync_copy(msg, acc.at[pl.ds(base + k * K, K)])
    plsc.subcore_barrier()

    ebase = (c * NS + s) * EPT
    pltpu.sync_copy(row.at[pl.ds(ebase, EPT)], row_all)
    pltpu.sync_copy(col.at[pl.ds(ebase, EPT)], col_all)

    def _body(g, carry):
        _edge_chunk_indices(row_all, col_all, rowbuf, colbuf, g * K)
        pltpu.async_copy(zp.at[rowbuf], msg, sem).wait()
        pltpu.sync_copy(msg, acc.at[colbuf], add=True)
        return carry

    lax.fori_loop(0, CHUNKS, _body, 0)
    plsc.subcore_barrier()
    pltpu.sync_copy(acc.at[pl.ds(base, RPT)], out.at[c, pl.ds(base, RPT)])


@functools.partial(
    pl.kernel,
    out_type=jax.ShapeDtypeStruct((NC, NPAD, 16), jnp.float32),
    mesh=_mesh,
    scratch_types=[
        pltpu.VMEM((K,), jnp.int32),
        pltpu.VMEM((K,), jnp.int32),
        pltpu.VMEM((EPT,), jnp.int32),
        pltpu.VMEM((EPT,), jnp.int32),
        pltpu.VMEM((K, 16), jnp.float32),
        pltpu.VMEM_SHARED((NPAD, 16), jnp.float32),
    ],
)
def _sc_deg(row, col, out, rowbuf, colbuf, row_all, col_all, buf, acc):
    c = lax.axis_index("c")
    s = lax.axis_index("s")

    def _fill(val):
        def _row(r, carry):
            buf[r, :] = jnp.full((16,), val, jnp.float32)
            return carry
        lax.fori_loop(0, K, _row, 0)

    _fill(0.0)
    base = s * RPT
    for k in range(RPT // K):
        pltpu.sync_copy(buf, acc.at[pl.ds(base + k * K, K)])
    plsc.subcore_barrier()
    _fill(1.0)

    ebase = (c * NS + s) * EPT
    pltpu.sync_copy(row.at[pl.ds(ebase, EPT)], row_all)
    pltpu.sync_copy(col.at[pl.ds(ebase, EPT)], col_all)

    def _body(g, carry):
        _edge_chunk_indices(row_all, col_all, rowbuf, colbuf, g * K)
        pltpu.sync_copy(buf, acc.at[colbuf], add=True)
        return carry

    lax.fori_loop(0, CHUNKS, _body, 0)
    plsc.subcore_barrier()
    pltpu.sync_copy(acc.at[pl.ds(base, RPT)], out.at[c, pl.ds(base, RPT)])


_RB = 400
_GRID = N // _RB


def _row_spec(width):
    return pl.BlockSpec((_RB, width), lambda i: (i, 0))


def _tc_first_body(x_ref, w_ref, d0_ref, d1_ref, zp_ref, dis_ref):
    deg = d0_ref[:, :1] + d1_ref[:, :1] + 1.0
    dis = lax.rsqrt(deg)
    z = jnp.dot(x_ref[...], w_ref[...], preferred_element_type=jnp.float32)
    zp_ref[...] = dis * z
    dis_ref[...] = jnp.broadcast_to(dis, (_RB, 16))


_tc_first = pl.pallas_call(
    _tc_first_body,
    grid=(_GRID,),
    in_specs=[
        _row_spec(D),
        pl.BlockSpec((D, D), lambda i: (0, 0)),
        _row_spec(16),
        _row_spec(16),
    ],
    out_specs=[_row_spec(D), _row_spec(16)],
    out_shape=[
        jax.ShapeDtypeStruct((N, D), jnp.float32),
        jax.ShapeDtypeStruct((N, 16), jnp.float32),
    ],
)


def _tc_mid_body(a0_ref, a1_ref, zp_ref, dis_ref, b_ref, w_ref, out_ref):
    dis = dis_ref[:, :1]
    h = dis * (a0_ref[...] + a1_ref[...] + zp_ref[...]) + b_ref[...]
    h = jnp.maximum(h, 0.0)
    out_ref[...] = dis * jnp.dot(
        h, w_ref[...], preferred_element_type=jnp.float32
    )


_tc_mid = pl.pallas_call(
    _tc_mid_body,
    grid=(_GRID,),
    in_specs=[
        _row_spec(D),
        _row_spec(D),
        _row_spec(D),
        _row_spec(16),
        pl.BlockSpec((1, D), lambda i: (0, 0)),
        pl.BlockSpec((D, D), lambda i: (0, 0)),
    ],
    out_specs=_row_spec(D),
    out_shape=jax.ShapeDtypeStruct((N, D), jnp.float32),
)


def _tc_last_body(a0_ref, a1_ref, zp_ref, dis_ref, b_ref, out_ref):
    dis = dis_ref[:, :1]
    out_ref[...] = dis * (a0_ref[...] + a1_ref[...] + zp_ref[...]) + b_ref[...]


_tc_last = pl.pallas_call(
    _tc_last_body,
    grid=(_GRID,),
    in_specs=[
        _row_spec(D),
        _row_spec(D),
        _row_spec(D),
        _row_spec(16),
        pl.BlockSpec((1, D), lambda i: (0, 0)),
    ],
    out_specs=_row_spec(D),
    out_shape=jax.ShapeDtypeStruct((N, D), jnp.float32),
)


def kernel(x, edge_index, W1, b1, W2, b2, W3, b3):
    row = edge_index[0]
    col = edge_index[1]

    degp = _sc_deg(row, col)
    z1p, dis16 = _tc_first(x, W1, degp[0, :N], degp[1, :N])

    a1 = _sc_agg(z1p, row, col)
    z2p = _tc_mid(a1[0, :N], a1[1, :N], z1p, dis16, b1.reshape(1, D), W2)

    a2 = _sc_agg(z2p, row, col)
    z3p = _tc_mid(a2[0, :N], a2[1, :N], z2p, dis16, b2.reshape(1, D), W3)

    a3 = _sc_agg(z3p, row, col)
    return _tc_last(a3[0, :N], a3[1, :N], z3p, dis16, b3.reshape(1, D))
